# Initial kernel scaffold; baseline (speedup 1.0000x reference)
import jax
import jax.numpy as jnp
from jax.experimental import pallas as pl


def kernel(x_scalar, x_rot, node_attr_scalar, node_attr_rot, edge_index, distance_embedding, rot, W1, b1, W2, b2):
    raise NotImplementedError("write your pallas kernel here")


# speedup vs baseline: 41.9510x; 41.9510x over previous
import functools

import jax
import jax.numpy as jnp
import numpy as np
from jax import lax
from jax.experimental import pallas as pl
from jax.experimental.pallas import tpu as pltpu
from jax.experimental.pallas import tpu_sc as plsc

_NS = 16
_NSA = 8
_NR = 4
_NRA = 2
_L = 2
_DE = 16
_HID = 64
_NREP = _NR + _NRA
_SC = _NS + _NSA
_RC = _NREP * _L * 2
_F = _SC + _RC
_MR = _NR * _L * 2


def _selection_matrices():
    P = np.zeros((2, _RC, _RC), np.float32)
    Q = np.zeros((2, 8, _RC), np.float32)
    Pb = np.zeros((2, _MR, _MR), np.float32)
    Qb = np.zeros((2, 8, _MR), np.float32)
    for m in range(2):
        for k in range(2):
            for l in range(2):
                for j in range(_NREP):
                    c = j * 4 + k * 2 + l
                    P[m, j * 4 + k * 2 + m, c] = 1.0
                    Q[m, k * 4 + l * 2 + m, c] = 1.0
                for j in range(_NR):
                    c = j * 4 + k * 2 + l
                    Pb[m, j * 4 + k * 2 + m, c] = 1.0
                    Qb[m, k * 4 + m * 2 + l, c] = 1.0
    return P, Q, Pb, Qb


_P, _Q, _PB, _QB = _selection_matrices()

_T_COLS = _HID + 4 * _RC + 2 * _RC + 2 * _MR


def _fuse_weights(W1, b1, W2, b2):
    A = jnp.zeros((2 * _F, _T_COLS), jnp.float32)
    A = A.at[0:_SC, 0:_HID].set(W1[0:_SC])
    A = A.at[_F:_F + _SC, 0:_HID].set(W1[_F:_F + _SC])
    c0 = _HID
    A = A.at[_SC:_F, c0:c0 + _RC].set(_P[0])
    A = A.at[_SC:_F, c0 + _RC:c0 + 2 * _RC].set(_P[1])
    A = A.at[_F + _SC:2 * _F, c0 + 2 * _RC:c0 + 3 * _RC].set(_P[0])
    A = A.at[_F + _SC:2 * _F, c0 + 3 * _RC:c0 + 4 * _RC].set(_P[1])
    B = jnp.zeros((_DE + 8, _T_COLS), jnp.float32)
    B = B.at[0:_DE, 0:_HID].set(W1[2 * _F:2 * _F + _DE])
    r0 = _HID + 4 * _RC
    B = B.at[_DE:, r0:r0 + _RC].set(jnp.asarray(_Q[0]))
    B = B.at[_DE:, r0 + _RC:r0 + 2 * _RC].set(jnp.asarray(_Q[1]))
    rb = r0 + 2 * _RC
    B = B.at[_DE:, rb:rb + _MR].set(jnp.asarray(_QB[0]))
    B = B.at[_DE:, rb + _MR:rb + 2 * _MR].set(jnp.asarray(_QB[1]))
    Wr = jnp.concatenate([W1[_SC:_F], W1[_F + _SC:2 * _F]], axis=0)
    W2e = jnp.concatenate(
        [W2[:, 0:_NS], W2[:, _NS:] @ _PB[0], W2[:, _NS:] @ _PB[1]], axis=1)
    b2e = jnp.concatenate([b2[0:_NS], b2[_NS:] @ _PB[0], b2[_NS:] @ _PB[1]])
    return A, B, Wr, W2e, b1.reshape(1, _HID), b2e.reshape(1, _NS + 2 * _MR)


def _mlp_body(x_ref, ed_ref, A_ref, B_ref, Wr_ref, W2_ref, b1_ref, b2_ref,
              os_ref, or_ref):
    x = x_ref[...]
    e = ed_ref[...]
    t = (jnp.dot(x, A_ref[...], preferred_element_type=jnp.float32)
         + jnp.dot(e, B_ref[...], preferred_element_type=jnp.float32))
    c0 = _HID
    hp = t[:, 0:_HID]
    Xd0 = t[:, c0:c0 + _RC]
    Xd1 = t[:, c0 + _RC:c0 + 2 * _RC]
    Xs0 = t[:, c0 + 2 * _RC:c0 + 3 * _RC]
    Xs1 = t[:, c0 + 3 * _RC:c0 + 4 * _RC]
    r0 = c0 + 4 * _RC
    R0 = t[:, r0:r0 + _RC]
    R1 = t[:, r0 + _RC:r0 + 2 * _RC]
    rb = r0 + 2 * _RC
    R0b = t[:, rb:rb + _MR]
    R1b = t[:, rb + _MR:rb + 2 * _MR]
    rotcat = jnp.concatenate([Xd0 * R0 + Xd1 * R1, Xs0 * R0 + Xs1 * R1], axis=1)
    h1 = hp + jnp.dot(rotcat, Wr_ref[...], preferred_element_type=jnp.float32) \
        + b1_ref[...]
    h = h1 * jax.nn.sigmoid(h1)
    u = jnp.dot(h, W2_ref[...], preferred_element_type=jnp.float32) + b2_ref[...]
    os_ref[...] = u[:, 0:_NS]
    or_ref[...] = (u[:, _NS:_NS + _MR] * R0b
                   + u[:, _NS + _MR:_NS + 2 * _MR] * R1b)


def _run_mlp(xcat, ed, A, B, Wr, W2e, b1r, b2r):
    E = xcat.shape[0]
    BE = 4000
    assert E % BE == 0
    full = lambda i: (0, 0)
    return pl.pallas_call(
        _mlp_body,
        grid=(E // BE,),
        in_specs=[
            pl.BlockSpec((BE, 2 * _F), lambda i: (i, 0)),
            pl.BlockSpec((BE, _DE + 8), lambda i: (i, 0)),
            pl.BlockSpec(A.shape, full),
            pl.BlockSpec(B.shape, full),
            pl.BlockSpec(Wr.shape, full),
            pl.BlockSpec(W2e.shape, full),
            pl.BlockSpec(b1r.shape, full),
            pl.BlockSpec(b2r.shape, full),
        ],
        out_specs=[
            pl.BlockSpec((BE, _NS), lambda i: (i, 0)),
            pl.BlockSpec((BE, _MR), lambda i: (i, 0)),
        ],
        out_shape=[
            jax.ShapeDtypeStruct((E, _NS), jnp.float32),
            jax.ShapeDtypeStruct((E, _MR), jnp.float32),
        ],
    )(xcat, ed, A, B, Wr, W2e, b1r, b2r)


_NC = 2
_NSUB = 16
_NW = _NC * _NSUB


def _gather_edges(feat, row, col):
    E = row.shape[0]
    N = feat.shape[0]
    EW = E // _NW
    C = 1000
    assert E % _NW == 0 and EW % C == 0
    mesh = plsc.VectorSubcoreMesh(core_axis_name="c", subcore_axis_name="s")

    @functools.partial(
        pl.kernel,
        out_type=jax.ShapeDtypeStruct((E, 2 * _F), jnp.float32),
        mesh=mesh,
        compiler_params=pltpu.CompilerParams(use_tc_tiling_on_sc=False),
        scratch_types=[
            pltpu.VMEM((C,), jnp.int32),
            pltpu.VMEM((C, _F), jnp.float32),
            pltpu.SemaphoreType.DMA,
        ],
    )
    def gather_k(feat_hbm, row_hbm, col_hbm, out_hbm, idx_v, rows_v, sem):
        wid = lax.axis_index("s") * _NC + lax.axis_index("c")
        base = wid * EW

        @pl.loop(0, EW, step=C)
        def _(off):
            e0 = base + off
            pltpu.sync_copy(col_hbm.at[pl.ds(e0, C)], idx_v)
            pltpu.async_copy(feat_hbm.at[idx_v], rows_v, sem).wait()
            pltpu.sync_copy(rows_v, out_hbm.at[pl.ds(e0, C), pl.ds(0, _F)])
            pltpu.sync_copy(row_hbm.at[pl.ds(e0, C)], idx_v)
            pltpu.async_copy(feat_hbm.at[idx_v], rows_v, sem).wait()
            pltpu.sync_copy(rows_v, out_hbm.at[pl.ds(e0, C), pl.ds(_F, _F)])

    return gather_k(feat, row, col)


def _scatter_messages(ms, mr, col, zeros, N):
    E = ms.shape[0]
    ET = E // _NSUB
    C = 1000
    RT = N // _NSUB
    assert E % _NSUB == 0 and ET % C == 0 and N % _NSUB == 0
    mesh = plsc.VectorSubcoreMesh(core_axis_name="c", subcore_axis_name="s")

    @functools.partial(
        pl.kernel,
        out_type=[
            jax.ShapeDtypeStruct((N, _NS), jnp.float32),
            jax.ShapeDtypeStruct((N, _MR), jnp.float32),
        ],
        mesh=mesh,
        compiler_params=pltpu.CompilerParams(use_tc_tiling_on_sc=False),
        scratch_types=[
            pltpu.VMEM((C,), jnp.int32),
            pltpu.VMEM((C, _NS), jnp.float32),
            pltpu.VMEM_SHARED((N, _NS), jnp.float32),
            pltpu.SemaphoreType.DMA,
        ],
    )
    def scatter_k(ms_hbm, mr_hbm, col_hbm, z_hbm, os_hbm, or_hbm,
                  idx_v, msg_v, acc, sem):
        cid = lax.axis_index("c")
        sid = lax.axis_index("s")
        pltpu.sync_copy(z_hbm.at[pl.ds(sid * RT, RT)],
                        acc.at[pl.ds(sid * RT, RT)])
        plsc.subcore_barrier()

        def accumulate(mess_hbm):
            @pl.loop(0, ET, step=C)
            def _(off):
                e0 = sid * ET + off
                pltpu.sync_copy(col_hbm.at[pl.ds(e0, C)], idx_v)
                pltpu.sync_copy(mess_hbm.at[pl.ds(e0, C)], msg_v)
                pltpu.sync_copy(msg_v, acc.at[idx_v], add=True)

        @pl.when(cid == 0)
        def _():
            accumulate(ms_hbm)

        @pl.when(cid == 1)
        def _():
            accumulate(mr_hbm)

        plsc.subcore_barrier()

        @pl.when(cid == 0)
        def _():
            pltpu.sync_copy(acc.at[pl.ds(sid * RT, RT)],
                            os_hbm.at[pl.ds(sid * RT, RT)])

        @pl.when(cid == 1)
        def _():
            pltpu.sync_copy(acc.at[pl.ds(sid * RT, RT)],
                            or_hbm.at[pl.ds(sid * RT, RT)])

    return scatter_k(ms, mr, col, zeros)


def kernel(x_scalar, x_rot, node_attr_scalar, node_attr_rot, edge_index,
           distance_embedding, rot, W1, b1, W2, b2):
    N = x_scalar.shape[0]
    E = edge_index.shape[1]
    feat = jnp.concatenate(
        [x_scalar, node_attr_scalar,
         x_rot.reshape(N, _NR * _L * 2), node_attr_rot.reshape(N, _NRA * _L * 2)],
        axis=1)
    ed = jnp.concatenate([distance_embedding, rot.reshape(E, 8)], axis=1)
    row = edge_index[0]
    col = edge_index[1]
    A, B, Wr, W2e, b1r, b2r = _fuse_weights(W1, b1, W2, b2)
    xcat = _gather_edges(feat, row, col)
    ms, mr = _run_mlp(xcat, ed, A, B, Wr, W2e, b1r, b2r)
    zeros = jnp.zeros((N, _NS), jnp.float32)
    out_scalar, out_rot = _scatter_messages(ms, mr, col, zeros, N)
    return out_scalar, out_rot.reshape(N, _NR, _L * 2)


# speedup vs baseline: 47.4136x; 1.1302x over previous
import functools

import jax
import jax.numpy as jnp
import numpy as np
from jax import lax
from jax.experimental import pallas as pl
from jax.experimental.pallas import tpu as pltpu
from jax.experimental.pallas import tpu_sc as plsc

_NS = 16
_NSA = 8
_NR = 4
_NRA = 2
_L = 2
_DE = 16
_HID = 64
_NREP = _NR + _NRA
_SC = _NS + _NSA
_RC = _NREP * _L * 2
_F = _SC + _RC
_MR = _NR * _L * 2


def _selection_matrices():
    P = np.zeros((2, _RC, _RC), np.float32)
    Q = np.zeros((2, 8, _RC), np.float32)
    Pb = np.zeros((2, _MR, _MR), np.float32)
    Qb = np.zeros((2, 8, _MR), np.float32)
    for m in range(2):
        for k in range(2):
            for l in range(2):
                for j in range(_NREP):
                    c = j * 4 + k * 2 + l
                    P[m, j * 4 + k * 2 + m, c] = 1.0
                    Q[m, k * 4 + l * 2 + m, c] = 1.0
                for j in range(_NR):
                    c = j * 4 + k * 2 + l
                    Pb[m, j * 4 + k * 2 + m, c] = 1.0
                    Qb[m, k * 4 + m * 2 + l, c] = 1.0
    return P, Q, Pb, Qb


_P, _Q, _PB, _QB = _selection_matrices()

def _fuse_weights(W1, b1, W2, b2):
    Ah = jnp.zeros((2 * _F, _HID), jnp.float32)
    Ah = Ah.at[0:_SC].set(W1[0:_SC])
    Ah = Ah.at[_F:_F + _SC].set(W1[_F:_F + _SC])
    P0 = jnp.zeros((2 * _F, 2 * _RC), jnp.float32)
    P1 = jnp.zeros((2 * _F, 2 * _RC), jnp.float32)
    P0 = P0.at[_SC:_F, 0:_RC].set(_P[0])
    P0 = P0.at[_F + _SC:2 * _F, _RC:2 * _RC].set(_P[0])
    P1 = P1.at[_SC:_F, 0:_RC].set(_P[1])
    P1 = P1.at[_F + _SC:2 * _F, _RC:2 * _RC].set(_P[1])
    Bh = W1[2 * _F:2 * _F + _DE]
    Q0 = jnp.concatenate([jnp.asarray(_Q[0])] * 2, axis=1)
    Q1 = jnp.concatenate([jnp.asarray(_Q[1])] * 2, axis=1)
    Qb0 = jnp.asarray(_QB[0])
    Qb1 = jnp.asarray(_QB[1])
    Wr = jnp.concatenate([W1[_SC:_F], W1[_F + _SC:2 * _F]], axis=0)
    W2s = W2[:, 0:_NS]
    W20 = W2[:, _NS:] @ _PB[0]
    W21 = W2[:, _NS:] @ _PB[1]
    b2s = b2[0:_NS].reshape(1, _NS)
    b20 = (b2[_NS:] @ _PB[0]).reshape(1, _MR)
    b21 = (b2[_NS:] @ _PB[1]).reshape(1, _MR)
    return (Ah, P0, P1, Bh, Q0, Q1, Qb0, Qb1, Wr, W2s, W20, W21,
            b1.reshape(1, _HID), b2s, b20, b21)


def _mlp_body(x_ref, de_ref, r_ref, Ah_ref, P0_ref, P1_ref, Bh_ref, Q0_ref,
              Q1_ref, Qb0_ref, Qb1_ref, Wr_ref, W2s_ref, W20_ref, W21_ref,
              b1_ref, b2s_ref, b20_ref, b21_ref, os_ref, or_ref):
    f32 = jnp.float32
    x = x_ref[...]
    de = de_ref[...]
    r = r_ref[...]
    hp = (jnp.dot(x, Ah_ref[...], preferred_element_type=f32)
          + jnp.dot(de, Bh_ref[...], preferred_element_type=f32)
          + b1_ref[...])
    X0 = jnp.dot(x, P0_ref[...], preferred_element_type=f32)
    X1 = jnp.dot(x, P1_ref[...], preferred_element_type=f32)
    R0 = jnp.dot(r, Q0_ref[...], preferred_element_type=f32)
    R1 = jnp.dot(r, Q1_ref[...], preferred_element_type=f32)
    rotcat = X0 * R0 + X1 * R1
    h1 = hp + jnp.dot(rotcat, Wr_ref[...], preferred_element_type=f32)
    h = h1 * jax.nn.sigmoid(h1)
    os_ref[...] = jnp.dot(h, W2s_ref[...], preferred_element_type=f32) \
        + b2s_ref[...]
    u0 = jnp.dot(h, W20_ref[...], preferred_element_type=f32) + b20_ref[...]
    u1 = jnp.dot(h, W21_ref[...], preferred_element_type=f32) + b21_ref[...]
    B0 = jnp.dot(r, Qb0_ref[...], preferred_element_type=f32)
    B1 = jnp.dot(r, Qb1_ref[...], preferred_element_type=f32)
    or_ref[...] = u0 * B0 + u1 * B1


def _run_mlp(xcat, de, r8, weights):
    E = xcat.shape[0]
    BE = 4000
    assert E % BE == 0
    full = lambda i: (0, 0)
    return pl.pallas_call(
        _mlp_body,
        grid=(E // BE,),
        in_specs=[
            pl.BlockSpec((BE, 2 * _F), lambda i: (i, 0)),
            pl.BlockSpec((BE, _DE), lambda i: (i, 0)),
            pl.BlockSpec((BE, 8), lambda i: (i, 0)),
        ] + [pl.BlockSpec(w.shape, full) for w in weights],
        out_specs=[
            pl.BlockSpec((BE, _NS), lambda i: (i, 0)),
            pl.BlockSpec((BE, _MR), lambda i: (i, 0)),
        ],
        out_shape=[
            jax.ShapeDtypeStruct((E, _NS), jnp.float32),
            jax.ShapeDtypeStruct((E, _MR), jnp.float32),
        ],
    )(xcat, de, r8, *weights)


_NC = 2
_NSUB = 16
_NW = _NC * _NSUB


def _gather_edges(feat, row, col):
    E = row.shape[0]
    N = feat.shape[0]
    EW = E // _NW
    C = 1000
    assert E % _NW == 0 and EW % C == 0
    mesh = plsc.VectorSubcoreMesh(core_axis_name="c", subcore_axis_name="s")

    @functools.partial(
        pl.kernel,
        out_type=jax.ShapeDtypeStruct((E, 2 * _F), jnp.float32),
        mesh=mesh,
        compiler_params=pltpu.CompilerParams(use_tc_tiling_on_sc=False),
        scratch_types=[
            pltpu.VMEM((C,), jnp.int32),
            pltpu.VMEM((C, _F), jnp.float32),
            pltpu.SemaphoreType.DMA,
        ],
    )
    def gather_k(feat_hbm, row_hbm, col_hbm, out_hbm, idx_v, rows_v, sem):
        wid = lax.axis_index("s") * _NC + lax.axis_index("c")
        base = wid * EW

        @pl.loop(0, EW, step=C)
        def _(off):
            e0 = base + off
            pltpu.sync_copy(col_hbm.at[pl.ds(e0, C)], idx_v)
            pltpu.async_copy(feat_hbm.at[idx_v], rows_v, sem).wait()
            pltpu.sync_copy(rows_v, out_hbm.at[pl.ds(e0, C), pl.ds(0, _F)])
            pltpu.sync_copy(row_hbm.at[pl.ds(e0, C)], idx_v)
            pltpu.async_copy(feat_hbm.at[idx_v], rows_v, sem).wait()
            pltpu.sync_copy(rows_v, out_hbm.at[pl.ds(e0, C), pl.ds(_F, _F)])

    return gather_k(feat, row, col)


def _scatter_messages(ms, mr, col, zeros, N):
    E = ms.shape[0]
    ET = E // _NSUB
    C = 1000
    RT = N // _NSUB
    assert E % _NSUB == 0 and ET % C == 0 and N % _NSUB == 0
    mesh = plsc.VectorSubcoreMesh(core_axis_name="c", subcore_axis_name="s")

    @functools.partial(
        pl.kernel,
        out_type=[
            jax.ShapeDtypeStruct((N, _NS), jnp.float32),
            jax.ShapeDtypeStruct((N, _MR), jnp.float32),
        ],
        mesh=mesh,
        compiler_params=pltpu.CompilerParams(use_tc_tiling_on_sc=False),
        scratch_types=[
            pltpu.VMEM((C,), jnp.int32),
            pltpu.VMEM((C, _NS), jnp.float32),
            pltpu.VMEM_SHARED((N, _NS), jnp.float32),
            pltpu.SemaphoreType.DMA,
        ],
    )
    def scatter_k(ms_hbm, mr_hbm, col_hbm, z_hbm, os_hbm, or_hbm,
                  idx_v, msg_v, acc, sem):
        cid = lax.axis_index("c")
        sid = lax.axis_index("s")
        pltpu.sync_copy(z_hbm.at[pl.ds(sid * RT, RT)],
                        acc.at[pl.ds(sid * RT, RT)])
        plsc.subcore_barrier()

        def accumulate(mess_hbm):
            @pl.loop(0, ET, step=C)
            def _(off):
                e0 = sid * ET + off
                pltpu.sync_copy(col_hbm.at[pl.ds(e0, C)], idx_v)
                pltpu.sync_copy(mess_hbm.at[pl.ds(e0, C)], msg_v)
                pltpu.sync_copy(msg_v, acc.at[idx_v], add=True)

        @pl.when(cid == 0)
        def _():
            accumulate(ms_hbm)

        @pl.when(cid == 1)
        def _():
            accumulate(mr_hbm)

        plsc.subcore_barrier()

        @pl.when(cid == 0)
        def _():
            pltpu.sync_copy(acc.at[pl.ds(sid * RT, RT)],
                            os_hbm.at[pl.ds(sid * RT, RT)])

        @pl.when(cid == 1)
        def _():
            pltpu.sync_copy(acc.at[pl.ds(sid * RT, RT)],
                            or_hbm.at[pl.ds(sid * RT, RT)])

    return scatter_k(ms, mr, col, zeros)


def kernel(x_scalar, x_rot, node_attr_scalar, node_attr_rot, edge_index,
           distance_embedding, rot, W1, b1, W2, b2):
    N = x_scalar.shape[0]
    E = edge_index.shape[1]
    feat = jnp.concatenate(
        [x_scalar, node_attr_scalar,
         x_rot.reshape(N, _NR * _L * 2), node_attr_rot.reshape(N, _NRA * _L * 2)],
        axis=1)
    r8 = rot.reshape(E, 8)
    row = edge_index[0]
    col = edge_index[1]
    weights = _fuse_weights(W1, b1, W2, b2)
    xcat = _gather_edges(feat, row, col)
    ms, mr = _run_mlp(xcat, distance_embedding, r8, weights)
    zeros = jnp.zeros((N, _NS), jnp.float32)
    out_scalar, out_rot = _scatter_messages(ms, mr, col, zeros, N)
    return out_scalar, out_rot.reshape(N, _NR, _L * 2)


# speedup vs baseline: 48.6792x; 1.0267x over previous
import functools

import jax
import jax.numpy as jnp
import numpy as np
from jax import lax
from jax.experimental import pallas as pl
from jax.experimental.pallas import tpu as pltpu
from jax.experimental.pallas import tpu_sc as plsc

_NS = 16
_NSA = 8
_NR = 4
_NRA = 2
_L = 2
_DE = 16
_HID = 64
_NREP = _NR + _NRA
_SC = _NS + _NSA
_RC = _NREP * _L * 2
_F = _SC + _RC
_MR = _NR * _L * 2


def _selection_matrices():
    P = np.zeros((2, _RC, _RC), np.float32)
    Q = np.zeros((2, 8, _RC), np.float32)
    Pb = np.zeros((2, _MR, _MR), np.float32)
    Qb = np.zeros((2, 8, _MR), np.float32)
    for m in range(2):
        for k in range(2):
            for l in range(2):
                for j in range(_NREP):
                    c = j * 4 + k * 2 + l
                    P[m, j * 4 + k * 2 + m, c] = 1.0
                    Q[m, k * 4 + l * 2 + m, c] = 1.0
                for j in range(_NR):
                    c = j * 4 + k * 2 + l
                    Pb[m, j * 4 + k * 2 + m, c] = 1.0
                    Qb[m, k * 4 + m * 2 + l, c] = 1.0
    return P, Q, Pb, Qb


_P, _Q, _PB, _QB = _selection_matrices()

def _fuse_weights(W1, b1, W2, b2):
    Ah = jnp.zeros((2 * _F, _HID), jnp.float32)
    Ah = Ah.at[0:_SC].set(W1[0:_SC])
    Ah = Ah.at[_F:_F + _SC].set(W1[_F:_F + _SC])
    P0 = jnp.zeros((2 * _F, 2 * _RC), jnp.float32)
    P1 = jnp.zeros((2 * _F, 2 * _RC), jnp.float32)
    P0 = P0.at[_SC:_F, 0:_RC].set(_P[0])
    P0 = P0.at[_F + _SC:2 * _F, _RC:2 * _RC].set(_P[0])
    P1 = P1.at[_SC:_F, 0:_RC].set(_P[1])
    P1 = P1.at[_F + _SC:2 * _F, _RC:2 * _RC].set(_P[1])
    Bh = W1[2 * _F:2 * _F + _DE]
    Q0 = jnp.concatenate([jnp.asarray(_Q[0])] * 2, axis=1)
    Q1 = jnp.concatenate([jnp.asarray(_Q[1])] * 2, axis=1)
    Qb0 = jnp.asarray(_QB[0])
    Qb1 = jnp.asarray(_QB[1])
    Wr = jnp.concatenate([W1[_SC:_F], W1[_F + _SC:2 * _F]], axis=0)
    W2s = W2[:, 0:_NS]
    W20 = W2[:, _NS:] @ _PB[0]
    W21 = W2[:, _NS:] @ _PB[1]
    b2s = b2[0:_NS].reshape(1, _NS)
    b20 = (b2[_NS:] @ _PB[0]).reshape(1, _MR)
    b21 = (b2[_NS:] @ _PB[1]).reshape(1, _MR)
    return (Ah, P0, P1, Bh, Q0, Q1, Qb0, Qb1, Wr, W2s, W20, W21,
            b1.reshape(1, _HID), b2s, b20, b21)


def _mlp_body(x_ref, de_ref, r_ref, Ah_ref, P0_ref, P1_ref, Bh_ref, Q0_ref,
              Q1_ref, Qb0_ref, Qb1_ref, Wr_ref, W2s_ref, W20_ref, W21_ref,
              b1_ref, b2s_ref, b20_ref, b21_ref, os_ref, or_ref):
    f32 = jnp.float32
    x = x_ref[...]
    de = de_ref[...]
    r = r_ref[...]
    hp = (jnp.dot(x, Ah_ref[...], preferred_element_type=f32)
          + jnp.dot(de, Bh_ref[...], preferred_element_type=f32)
          + b1_ref[...])
    X0 = jnp.dot(x, P0_ref[...], preferred_element_type=f32)
    X1 = jnp.dot(x, P1_ref[...], preferred_element_type=f32)
    R0 = jnp.dot(r, Q0_ref[...], preferred_element_type=f32)
    R1 = jnp.dot(r, Q1_ref[...], preferred_element_type=f32)
    rotcat = X0 * R0 + X1 * R1
    h1 = hp + jnp.dot(rotcat, Wr_ref[...], preferred_element_type=f32)
    h = h1 * jax.nn.sigmoid(h1)
    os_ref[...] = jnp.dot(h, W2s_ref[...], preferred_element_type=f32) \
        + b2s_ref[...]
    u0 = jnp.dot(h, W20_ref[...], preferred_element_type=f32) + b20_ref[...]
    u1 = jnp.dot(h, W21_ref[...], preferred_element_type=f32) + b21_ref[...]
    B0 = jnp.dot(r, Qb0_ref[...], preferred_element_type=f32)
    B1 = jnp.dot(r, Qb1_ref[...], preferred_element_type=f32)
    or_ref[...] = u0 * B0 + u1 * B1


def _run_mlp(xcat, de, r8, weights):
    E = xcat.shape[0]
    BE = 8000
    assert E % BE == 0
    full = lambda i: (0, 0)
    return pl.pallas_call(
        _mlp_body,
        grid=(E // BE,),
        in_specs=[
            pl.BlockSpec((BE, 2 * _F), lambda i: (i, 0)),
            pl.BlockSpec((BE, _DE), lambda i: (i, 0)),
            pl.BlockSpec((BE, 8), lambda i: (i, 0)),
        ] + [pl.BlockSpec(w.shape, full) for w in weights],
        out_specs=[
            pl.BlockSpec((BE, _NS), lambda i: (i, 0)),
            pl.BlockSpec((BE, _MR), lambda i: (i, 0)),
        ],
        out_shape=[
            jax.ShapeDtypeStruct((E, _NS), jnp.float32),
            jax.ShapeDtypeStruct((E, _MR), jnp.float32),
        ],
    )(xcat, de, r8, *weights)


_NC = 2
_NSUB = 16
_NW = _NC * _NSUB


def _gather_edges(feat, row, col):
    E = row.shape[0]
    N = feat.shape[0]
    EW = E // _NW
    C = 1000
    assert E % _NW == 0 and EW % C == 0
    mesh = plsc.VectorSubcoreMesh(core_axis_name="c", subcore_axis_name="s")

    @functools.partial(
        pl.kernel,
        out_type=jax.ShapeDtypeStruct((E, 2 * _F), jnp.float32),
        mesh=mesh,
        compiler_params=pltpu.CompilerParams(use_tc_tiling_on_sc=False),
        scratch_types=[
            pltpu.VMEM((C,), jnp.int32),
            pltpu.VMEM((C, _F), jnp.float32),
            pltpu.SemaphoreType.DMA,
        ],
    )
    def gather_k(feat_hbm, row_hbm, col_hbm, out_hbm, idx_v, rows_v, sem):
        wid = lax.axis_index("s") * _NC + lax.axis_index("c")
        base = wid * EW

        @pl.loop(0, EW, step=C)
        def _(off):
            e0 = base + off
            pltpu.sync_copy(col_hbm.at[pl.ds(e0, C)], idx_v)
            pltpu.async_copy(feat_hbm.at[idx_v], rows_v, sem).wait()
            pltpu.sync_copy(rows_v, out_hbm.at[pl.ds(e0, C), pl.ds(0, _F)])
            pltpu.sync_copy(row_hbm.at[pl.ds(e0, C)], idx_v)
            pltpu.async_copy(feat_hbm.at[idx_v], rows_v, sem).wait()
            pltpu.sync_copy(rows_v, out_hbm.at[pl.ds(e0, C), pl.ds(_F, _F)])

    return gather_k(feat, row, col)


def _scatter_messages(ms, mr, col, zeros, N):
    E = ms.shape[0]
    ET = E // _NSUB
    C = 1000
    RT = N // _NSUB
    assert E % _NSUB == 0 and ET % C == 0 and N % _NSUB == 0
    mesh = plsc.VectorSubcoreMesh(core_axis_name="c", subcore_axis_name="s")

    @functools.partial(
        pl.kernel,
        out_type=[
            jax.ShapeDtypeStruct((N, _NS), jnp.float32),
            jax.ShapeDtypeStruct((N, _MR), jnp.float32),
        ],
        mesh=mesh,
        compiler_params=pltpu.CompilerParams(use_tc_tiling_on_sc=False),
        scratch_types=[
            pltpu.VMEM((C,), jnp.int32),
            pltpu.VMEM((C, _NS), jnp.float32),
            pltpu.VMEM_SHARED((N, _NS), jnp.float32),
            pltpu.SemaphoreType.DMA,
        ],
    )
    def scatter_k(ms_hbm, mr_hbm, col_hbm, z_hbm, os_hbm, or_hbm,
                  idx_v, msg_v, acc, sem):
        cid = lax.axis_index("c")
        sid = lax.axis_index("s")
        pltpu.sync_copy(z_hbm.at[pl.ds(sid * RT, RT)],
                        acc.at[pl.ds(sid * RT, RT)])
        plsc.subcore_barrier()

        def accumulate(mess_hbm):
            @pl.loop(0, ET, step=C)
            def _(off):
                e0 = sid * ET + off
                pltpu.sync_copy(col_hbm.at[pl.ds(e0, C)], idx_v)
                pltpu.sync_copy(mess_hbm.at[pl.ds(e0, C)], msg_v)
                pltpu.sync_copy(msg_v, acc.at[idx_v], add=True)

        @pl.when(cid == 0)
        def _():
            accumulate(ms_hbm)

        @pl.when(cid == 1)
        def _():
            accumulate(mr_hbm)

        plsc.subcore_barrier()

        @pl.when(cid == 0)
        def _():
            pltpu.sync_copy(acc.at[pl.ds(sid * RT, RT)],
                            os_hbm.at[pl.ds(sid * RT, RT)])

        @pl.when(cid == 1)
        def _():
            pltpu.sync_copy(acc.at[pl.ds(sid * RT, RT)],
                            or_hbm.at[pl.ds(sid * RT, RT)])

    return scatter_k(ms, mr, col, zeros)


def kernel(x_scalar, x_rot, node_attr_scalar, node_attr_rot, edge_index,
           distance_embedding, rot, W1, b1, W2, b2):
    N = x_scalar.shape[0]
    E = edge_index.shape[1]
    feat = jnp.concatenate(
        [x_scalar, node_attr_scalar,
         x_rot.reshape(N, _NR * _L * 2), node_attr_rot.reshape(N, _NRA * _L * 2)],
        axis=1)
    r8 = rot.reshape(E, 8)
    row = edge_index[0]
    col = edge_index[1]
    weights = _fuse_weights(W1, b1, W2, b2)
    xcat = _gather_edges(feat, row, col)
    ms, mr = _run_mlp(xcat, distance_embedding, r8, weights)
    zeros = jnp.zeros((N, _NS), jnp.float32)
    out_scalar, out_rot = _scatter_messages(ms, mr, col, zeros, N)
    return out_scalar, out_rot.reshape(N, _NR, _L * 2)


# speedup vs baseline: 63.2367x; 1.2991x over previous
import functools

import jax
import jax.numpy as jnp
import numpy as np
from jax import lax
from jax.experimental import pallas as pl
from jax.experimental.pallas import tpu as pltpu
from jax.experimental.pallas import tpu_sc as plsc

_NS = 16
_NSA = 8
_NR = 4
_NRA = 2
_L = 2
_DE = 16
_HID = 64
_NREP = _NR + _NRA
_SC = _NS + _NSA
_RC = _NREP * _L * 2
_F = _SC + _RC
_MR = _NR * _L * 2


def _selection_matrices():
    P = np.zeros((2, _RC, _RC), np.float32)
    Q = np.zeros((2, 8, _RC), np.float32)
    Pb = np.zeros((2, _MR, _MR), np.float32)
    Qb = np.zeros((2, 8, _MR), np.float32)
    for m in range(2):
        for k in range(2):
            for l in range(2):
                for j in range(_NREP):
                    c = j * 4 + k * 2 + l
                    P[m, j * 4 + k * 2 + m, c] = 1.0
                    Q[m, k * 4 + l * 2 + m, c] = 1.0
                for j in range(_NR):
                    c = j * 4 + k * 2 + l
                    Pb[m, j * 4 + k * 2 + m, c] = 1.0
                    Qb[m, k * 4 + m * 2 + l, c] = 1.0
    return P, Q, Pb, Qb


_P, _Q, _PB, _QB = _selection_matrices()

_FP = 64
_ED = _DE + 8


def _fuse_weights(W1, b1, W2, b2):
    Ah = jnp.zeros((2 * _FP, _HID), jnp.float32)
    Ah = Ah.at[0:_SC].set(W1[0:_SC])
    Ah = Ah.at[_FP:_FP + _SC].set(W1[_F:_F + _SC])
    P0 = jnp.zeros((2 * _FP, 2 * _RC), jnp.float32)
    P1 = jnp.zeros((2 * _FP, 2 * _RC), jnp.float32)
    P0 = P0.at[_SC:_F, 0:_RC].set(_P[0])
    P0 = P0.at[_FP + _SC:_FP + _F, _RC:2 * _RC].set(_P[0])
    P1 = P1.at[_SC:_F, 0:_RC].set(_P[1])
    P1 = P1.at[_FP + _SC:_FP + _F, _RC:2 * _RC].set(_P[1])
    Bh = jnp.zeros((_ED, _HID), jnp.float32)
    Bh = Bh.at[0:_DE].set(W1[2 * _F:2 * _F + _DE])
    Q0 = jnp.zeros((_ED, 2 * _RC), jnp.float32)
    Q1 = jnp.zeros((_ED, 2 * _RC), jnp.float32)
    Q0 = Q0.at[_DE:].set(jnp.concatenate([jnp.asarray(_Q[0])] * 2, axis=1))
    Q1 = Q1.at[_DE:].set(jnp.concatenate([jnp.asarray(_Q[1])] * 2, axis=1))
    Qb0 = jnp.zeros((_ED, _NS + _MR), jnp.float32)
    Qb1 = jnp.zeros((_ED, _NS + _MR), jnp.float32)
    Qb0 = Qb0.at[_DE:, _NS:].set(jnp.asarray(_QB[0]))
    Qb1 = Qb1.at[_DE:, _NS:].set(jnp.asarray(_QB[1]))
    Wr = jnp.concatenate([W1[_SC:_F], W1[_F + _SC:2 * _F]], axis=0)
    W2s = jnp.zeros((_HID, _NS + _MR), jnp.float32)
    W2s = W2s.at[:, 0:_NS].set(W2[:, 0:_NS])
    W20 = jnp.zeros((_HID, _NS + _MR), jnp.float32)
    W21 = jnp.zeros((_HID, _NS + _MR), jnp.float32)
    W20 = W20.at[:, _NS:].set(W2[:, _NS:] @ _PB[0])
    W21 = W21.at[:, _NS:].set(W2[:, _NS:] @ _PB[1])
    b2s = jnp.zeros((1, _NS + _MR), jnp.float32).at[0, 0:_NS].set(b2[0:_NS])
    b20 = jnp.zeros((1, _NS + _MR), jnp.float32).at[0, _NS:].set(b2[_NS:] @ _PB[0])
    b21 = jnp.zeros((1, _NS + _MR), jnp.float32).at[0, _NS:].set(b2[_NS:] @ _PB[1])
    return (Ah, P0, P1, Bh, Q0, Q1, Qb0, Qb1, Wr, W2s, W20, W21,
            b1.reshape(1, _HID), b2s, b20, b21)


def _mlp_body(x_ref, ed_ref, Ah_ref, P0_ref, P1_ref, Bh_ref, Q0_ref,
              Q1_ref, Qb0_ref, Qb1_ref, Wr_ref, W2s_ref, W20_ref, W21_ref,
              b1_ref, b2s_ref, b20_ref, b21_ref, out_ref):
    f32 = jnp.float32
    x = x_ref[...]
    ed = ed_ref[...]
    hp = (jnp.dot(x, Ah_ref[...], preferred_element_type=f32)
          + jnp.dot(ed, Bh_ref[...], preferred_element_type=f32)
          + b1_ref[...])
    X0 = jnp.dot(x, P0_ref[...], preferred_element_type=f32)
    X1 = jnp.dot(x, P1_ref[...], preferred_element_type=f32)
    R0 = jnp.dot(ed, Q0_ref[...], preferred_element_type=f32)
    R1 = jnp.dot(ed, Q1_ref[...], preferred_element_type=f32)
    rotcat = X0 * R0 + X1 * R1
    h1 = hp + jnp.dot(rotcat, Wr_ref[...], preferred_element_type=f32)
    h = h1 * jax.nn.sigmoid(h1)
    u0 = jnp.dot(h, W20_ref[...], preferred_element_type=f32) + b20_ref[...]
    u1 = jnp.dot(h, W21_ref[...], preferred_element_type=f32) + b21_ref[...]
    B0 = jnp.dot(ed, Qb0_ref[...], preferred_element_type=f32)
    B1 = jnp.dot(ed, Qb1_ref[...], preferred_element_type=f32)
    out_ref[...] = (jnp.dot(h, W2s_ref[...], preferred_element_type=f32)
                    + b2s_ref[...] + u0 * B0 + u1 * B1)


def _run_mlp(xcat, ed, weights):
    E = xcat.shape[0]
    BE = 8000
    assert E % BE == 0
    full = lambda i: (0, 0)
    return pl.pallas_call(
        _mlp_body,
        grid=(E // BE,),
        in_specs=[
            pl.BlockSpec((BE, 2 * _FP), lambda i: (i, 0)),
            pl.BlockSpec((BE, _ED), lambda i: (i, 0)),
        ] + [pl.BlockSpec(w.shape, full) for w in weights],
        out_specs=pl.BlockSpec((BE, _NS + _MR), lambda i: (i, 0)),
        out_shape=jax.ShapeDtypeStruct((E, _NS + _MR), jnp.float32),
    )(xcat, ed, *weights)


_NC = 2
_NSUB = 16
_NW = _NC * _NSUB


def _gather_edges(feat, row, col):
    E = row.shape[0]
    EW = E // _NW
    C = 1000
    assert E % _NW == 0 and EW % C == 0
    mesh = plsc.VectorSubcoreMesh(core_axis_name="c", subcore_axis_name="s")

    @functools.partial(
        pl.kernel,
        out_type=jax.ShapeDtypeStruct((E, 2 * _FP), jnp.float32),
        mesh=mesh,
        compiler_params=pltpu.CompilerParams(use_tc_tiling_on_sc=False),
        scratch_types=[
            pltpu.VMEM((C,), jnp.int32),
            pltpu.VMEM((C, _FP), jnp.float32),
            pltpu.SemaphoreType.DMA,
        ],
    )
    def gather_k(feat_hbm, row_hbm, col_hbm, out_hbm, idx_v, rows_v, sem):
        wid = lax.axis_index("s") * _NC + lax.axis_index("c")
        base = wid * EW

        @pl.loop(0, EW, step=C)
        def _(off):
            e0 = base + off
            pltpu.sync_copy(col_hbm.at[pl.ds(e0, C)], idx_v)
            pltpu.async_copy(feat_hbm.at[idx_v], rows_v, sem).wait()
            pltpu.sync_copy(rows_v, out_hbm.at[pl.ds(e0, C), pl.ds(0, _FP)])
            pltpu.sync_copy(row_hbm.at[pl.ds(e0, C)], idx_v)
            pltpu.async_copy(feat_hbm.at[idx_v], rows_v, sem).wait()
            pltpu.sync_copy(rows_v, out_hbm.at[pl.ds(e0, C), pl.ds(_FP, _FP)])

    return gather_k(feat, row, col)


def _scatter_messages(m32, col, zeros, N):
    E = m32.shape[0]
    ET = E // _NSUB
    C = 1000
    RT = N // _NSUB
    assert E % _NSUB == 0 and ET % C == 0 and N % _NSUB == 0
    mesh = plsc.VectorSubcoreMesh(core_axis_name="c", subcore_axis_name="s")

    @functools.partial(
        pl.kernel,
        out_type=[
            jax.ShapeDtypeStruct((N, _NS), jnp.float32),
            jax.ShapeDtypeStruct((N, _MR), jnp.float32),
        ],
        mesh=mesh,
        compiler_params=pltpu.CompilerParams(use_tc_tiling_on_sc=False),
        scratch_types=[
            pltpu.VMEM((C,), jnp.int32),
            pltpu.VMEM((C, _NS), jnp.float32),
            pltpu.VMEM_SHARED((N, _NS), jnp.float32),
            pltpu.SemaphoreType.DMA,
        ],
    )
    def scatter_k(m_hbm, col_hbm, z_hbm, os_hbm, or_hbm,
                  idx_v, msg_v, acc, sem):
        cid = lax.axis_index("c")
        sid = lax.axis_index("s")
        pltpu.sync_copy(z_hbm.at[pl.ds(sid * RT, RT)],
                        acc.at[pl.ds(sid * RT, RT)])
        plsc.subcore_barrier()

        @pl.loop(0, ET, step=C)
        def _(off):
            e0 = sid * ET + off
            pltpu.sync_copy(col_hbm.at[pl.ds(e0, C)], idx_v)
            pltpu.sync_copy(m_hbm.at[pl.ds(e0, C), pl.ds(cid * _NS, _NS)],
                            msg_v)
            pltpu.sync_copy(msg_v, acc.at[idx_v], add=True)

        plsc.subcore_barrier()

        @pl.when(cid == 0)
        def _():
            pltpu.sync_copy(acc.at[pl.ds(sid * RT, RT)],
                            os_hbm.at[pl.ds(sid * RT, RT)])

        @pl.when(cid == 1)
        def _():
            pltpu.sync_copy(acc.at[pl.ds(sid * RT, RT)],
                            or_hbm.at[pl.ds(sid * RT, RT)])

    return scatter_k(m32, col, zeros)


def kernel(x_scalar, x_rot, node_attr_scalar, node_attr_rot, edge_index,
           distance_embedding, rot, W1, b1, W2, b2):
    N = x_scalar.shape[0]
    E = edge_index.shape[1]
    feat = jnp.concatenate(
        [x_scalar, node_attr_scalar,
         x_rot.reshape(N, _NR * _L * 2), node_attr_rot.reshape(N, _NRA * _L * 2),
         jnp.zeros((N, _FP - _F), jnp.float32)],
        axis=1)
    ed = jnp.concatenate([distance_embedding, rot.reshape(E, 8)], axis=1)
    row = edge_index[0]
    col = edge_index[1]
    weights = _fuse_weights(W1, b1, W2, b2)
    xcat = _gather_edges(feat, row, col)
    m32 = _run_mlp(xcat, ed, weights)
    zeros = jnp.zeros((N, _NS), jnp.float32)
    out_scalar, out_rot = _scatter_messages(m32, col, zeros, N)
    return out_scalar, out_rot.reshape(N, _NR, _L * 2)
